# baseline (speedup 1.0000x reference)
import functools

import jax
import jax.numpy as jnp
from jax import lax
from jax.experimental import pallas as pl
from jax.experimental.pallas import tpu as pltpu
from jax.experimental.pallas import tpu_sc as plsc

N = 10000
E = 160000
DN = 128
DE = 16
H = 64
EH = 128
RH = 1000
STEPS = 6

NC = 2
NS = 16
NW = NC * NS
CHUNK = 128
EPW = E // NW
CPW = 40
TAIL = EPW - (CPW - 1) * CHUNK
NPAD = N + 8
OUT_ROWS = 640

@functools.cache
def _sc_mesh():
    return plsc.VectorSubcoreMesh(core_axis_name="c", subcore_axis_name="s",
                                  num_cores=NC, num_subcores=NS)


@functools.cache
def _sc_gather_call():
    @functools.partial(
        pl.kernel,
        out_type=jax.ShapeDtypeStruct((E, H), jnp.float32),
        mesh=_sc_mesh(),
        scratch_types=[
            pltpu.VMEM((CPW, CHUNK), jnp.int32),
            pltpu.VMEM((CHUNK, H), jnp.float32),
            pltpu.SemaphoreType.DMA,
        ],
        compiler_params=pltpu.CompilerParams(use_tc_tiling_on_sc=False),
    )
    def _sc_gather(h_hbm, idx_hbm, out_hbm, idx_v, rows_v, sem):
        c = lax.axis_index("c")
        s = lax.axis_index("s")
        wid = s * NC + c
        pltpu.sync_copy(idx_hbm.at[pl.ds(wid * CPW, CPW)], idx_v)
        for j in range(CPW - 1):
            pltpu.async_copy(h_hbm.at[idx_v.at[j]], rows_v, sem).wait()
            pltpu.sync_copy(rows_v,
                            out_hbm.at[pl.ds(wid * EPW + j * CHUNK, CHUNK)])
        pltpu.async_copy(h_hbm.at[idx_v.at[CPW - 1]], rows_v, sem).wait()
        pltpu.sync_copy(
            rows_v.at[pl.ds(0, TAIL)],
            out_hbm.at[pl.ds(wid * EPW + (CPW - 1) * CHUNK, TAIL)])

    return _sc_gather


@functools.cache
def _sc_scatter_call():
    @functools.partial(
        pl.kernel,
        out_type=jax.ShapeDtypeStruct((NC, N, H), jnp.float32),
        mesh=_sc_mesh(),
        scratch_types=[
            pltpu.VMEM((CPW, CHUNK), jnp.int32),
            pltpu.VMEM((CHUNK, H), jnp.float32),
            pltpu.VMEM_SHARED((NPAD, H), jnp.float32),
        ],
        compiler_params=pltpu.CompilerParams(use_tc_tiling_on_sc=False),
    )
    def _sc_scatter(m_hbm, idx_hbm, zeros_hbm, out_hbm, idx_v, m_v, agg_sh):
        c = lax.axis_index("c")
        s = lax.axis_index("s")
        wid = s * NC + c

        @pl.when(s == 0)
        def _():
            pltpu.sync_copy(zeros_hbm, agg_sh)

        plsc.subcore_barrier()
        pltpu.sync_copy(idx_hbm.at[pl.ds(wid * CPW, CPW)], idx_v)
        for j in range(CPW - 1):
            pltpu.sync_copy(m_hbm.at[pl.ds(wid * EPW + j * CHUNK, CHUNK)], m_v)
            pltpu.sync_copy(m_v, agg_sh.at[idx_v.at[j]], add=True)
        pltpu.sync_copy(
            m_hbm.at[pl.ds(wid * EPW + (CPW - 1) * CHUNK, TAIL)],
            m_v.at[pl.ds(0, TAIL)])
        pltpu.sync_copy(m_v, agg_sh.at[idx_v.at[CPW - 1]], add=True)
        plsc.subcore_barrier()
        off = jnp.minimum(s * OUT_ROWS, N - OUT_ROWS)
        pltpu.sync_copy(agg_sh.at[pl.ds(off, OUT_ROWS)],
                        out_hbm.at[c].at[pl.ds(off, OUT_ROWS)])

    return _sc_scatter


def _mm_kernel(x_ref, w_ref, b_ref, o_ref, *, act, out_dtype):
    y = jnp.dot(x_ref[...], w_ref[...], preferred_element_type=jnp.float32, precision=jax.lax.Precision.HIGHEST)
    y = y + b_ref[...]
    if act == "relu":
        y = jnp.maximum(y, 0.0)
    o_ref[...] = y.astype(out_dtype)


def _dense(x, w, b, act, gridn, out_dtype=jnp.float32):
    tm = x.shape[0] // gridn
    return pl.pallas_call(
        functools.partial(_mm_kernel, act=act, out_dtype=out_dtype),
        grid=(gridn,),
        in_specs=[
            pl.BlockSpec((tm, x.shape[1]), lambda i: (i, 0)),
            pl.BlockSpec(w.shape, lambda i: (0, 0)),
            pl.BlockSpec(b.shape, lambda i: (0, 0)),
        ],
        out_specs=pl.BlockSpec((tm, w.shape[1]), lambda i: (i, 0)),
        out_shape=jax.ShapeDtypeStruct((x.shape[0], w.shape[1]), out_dtype),
    )(x, w, b)


TE_MSG = 1280


def _msg_kernel(hs_ref, w_ref, o_ref):
    hs = hs_ref[...]
    w = w_ref[...]
    hs_a = hs[:, :32][:, :, None]
    hs_b = hs[:, 32:][:, :, None]
    wl = w[:, :, :64].astype(jnp.float32)
    wr = w[:, :, 64:].astype(jnp.float32)
    o_ref[...] = jnp.sum(wl * hs_a, axis=1) + jnp.sum(wr * hs_b, axis=1)


def _messages(hs, wedge3):
    return pl.pallas_call(
        _msg_kernel,
        grid=(E // TE_MSG,),
        in_specs=[
            pl.BlockSpec((TE_MSG, H), lambda i: (i, 0)),
            pl.BlockSpec((TE_MSG, 32, 128), lambda i: (i, 0, 0)),
        ],
        out_specs=pl.BlockSpec((TE_MSG, H), lambda i: (i, 0)),
        out_shape=jax.ShapeDtypeStruct((E, H), jnp.float32),
    )(hs, wedge3)


TM_WEDGE = 1000


def _wedge_kernel(x_ref, w_ref, b_ref, o_ref):
    y = jnp.dot(x_ref[...], w_ref[...], preferred_element_type=jnp.float32,
                precision=jax.lax.Precision.DEFAULT)
    y = (y + b_ref[...]).astype(jnp.bfloat16)
    for p in range(32):
        o_ref[:, p, :] = y[:, 128 * p:128 * (p + 1)]


def _wedge_build(eh, w, b):
    return pl.pallas_call(
        _wedge_kernel,
        grid=(E // TM_WEDGE,),
        in_specs=[
            pl.BlockSpec((TM_WEDGE, EH), lambda i: (i, 0)),
            pl.BlockSpec((EH, H * H), lambda i: (0, 0)),
            pl.BlockSpec((1, H * H), lambda i: (0, 0)),
        ],
        out_specs=pl.BlockSpec((TM_WEDGE, 32, 128), lambda i: (i, 0, 0)),
        out_shape=jax.ShapeDtypeStruct((E, 32, 128), jnp.bfloat16),
    )(eh, w, b)


def _gru_kernel(agg_ref, cb_ref, hid_ref, wir_ref, wiz_ref, win_ref,
                whr_ref, whz_ref, whn_ref, bir_ref, biz_ref, bin_ref,
                bhr_ref, bhz_ref, bhn_ref, o_ref):
    a = jnp.maximum(agg_ref[0] + agg_ref[1] + cb_ref[...], 0.0)
    hid = hid_ref[...]

    def mm(x, wref):
        return jnp.dot(x, wref[...], preferred_element_type=jnp.float32, precision=jax.lax.Precision.HIGHEST)

    r = jax.nn.sigmoid(mm(a, wir_ref) + bir_ref[...] + mm(hid, whr_ref)
                       + bhr_ref[...])
    z = jax.nn.sigmoid(mm(a, wiz_ref) + biz_ref[...] + mm(hid, whz_ref)
                       + bhz_ref[...])
    n = jnp.tanh(mm(a, win_ref) + bin_ref[...]
                 + r * (mm(hid, whn_ref) + bhn_ref[...]))
    o_ref[...] = (1.0 - z) * n + z * hid


TN_GRU = 1000


def _gru(agg2, cb, hid, wmats, bvecs):
    full = lambda shape: pl.BlockSpec(shape, lambda i: (0,) * len(shape))
    return pl.pallas_call(
        _gru_kernel,
        grid=(N // TN_GRU,),
        in_specs=[
            pl.BlockSpec((NC, TN_GRU, H), lambda i: (0, i, 0)),
            full((1, H)),
            pl.BlockSpec((TN_GRU, H), lambda i: (i, 0)),
        ] + [full((H, H))] * 6 + [full((1, H))] * 6,
        out_specs=pl.BlockSpec((TN_GRU, H), lambda i: (i, 0)),
        out_shape=jax.ShapeDtypeStruct((N, H), jnp.float32),
    )(agg2, cb, hid, *wmats, *bvecs)


def _readout_kernel(h_ref, w1_ref, b1_ref, w2r_ref, b2_ref, o_ref):
    i = pl.program_id(0)
    x = jnp.dot(h_ref[...], w1_ref[...], preferred_element_type=jnp.float32, precision=jax.lax.Precision.HIGHEST)
    x = x + b1_ref[...]
    x = jnp.where(x > 0, x, jnp.exp(jnp.minimum(x, 0.0)) - 1.0)
    part = jnp.sum(x * w2r_ref[...]) + TN_GRU * jnp.sum(b2_ref[...])

    @pl.when(i == 0)
    def _():
        o_ref[...] = jnp.zeros((1, 1), jnp.float32)

    o_ref[...] += jnp.full((1, 1), part)


def _readout(h, w1, b1, w2r, b2):
    full = lambda shape: pl.BlockSpec(shape, lambda i: (0,) * len(shape))
    return pl.pallas_call(
        _readout_kernel,
        grid=(N // TN_GRU,),
        in_specs=[
            pl.BlockSpec((TN_GRU, H), lambda i: (i, 0)),
            full((H, RH)),
            full((1, RH)),
            full((1, RH)),
            full((1, 1)),
        ],
        out_specs=pl.BlockSpec((1, 1), lambda i: (0, 0)),
        out_shape=jax.ShapeDtypeStruct((1, 1), jnp.float32),
    )(h, w1, b1, w2r, b2)


def kernel(edge_index, nodes, edges, W0, b0, We1, be1, We2, be2, conv_bias,
           W_ih, W_hh, b_ih, b_hh, Wr1, br1, Wr2, br2):
    f32 = jnp.float32
    padc = CPW * CHUNK - EPW
    src = jnp.pad(edge_index[0].reshape(NW, EPW), ((0, 0), (0, padc)),
                  constant_values=0).reshape(NW * CPW, CHUNK)
    dst = jnp.pad(edge_index[1].reshape(NW, EPW), ((0, 0), (0, padc)),
                  constant_values=N).reshape(NW * CPW, CHUNK)

    h0 = _dense(nodes, W0, b0.reshape(1, H), "relu", 10)
    eh = _dense(edges, We1, be1.reshape(1, EH), "relu", 40)
    j = jnp.arange(H * H)
    p_idx = j // 128
    q_idx = j % 128
    perm = (p_idx + 32 * (q_idx >= 64)) * H + (q_idx % 64)
    We2p = We2[:, perm]
    be2p = be2[perm]
    wedge3 = _wedge_build(eh, We2p, be2p.reshape(1, H * H))

    zeros = jnp.zeros((NPAD, H), f32)
    cb = conv_bias.reshape(1, H)
    wmats = [W_ih[:H].T, W_ih[H:2 * H].T, W_ih[2 * H:].T,
             W_hh[:H].T, W_hh[H:2 * H].T, W_hh[2 * H:].T]
    bvecs = [b_ih[:H].reshape(1, H), b_ih[H:2 * H].reshape(1, H),
             b_ih[2 * H:].reshape(1, H), b_hh[:H].reshape(1, H),
             b_hh[H:2 * H].reshape(1, H), b_hh[2 * H:].reshape(1, H)]

    h = h0
    hid = h0
    for _ in range(STEPS):
        hs = _sc_gather_call()(h, src)
        m = _messages(hs, wedge3)
        agg2 = _sc_scatter_call()(m, dst, zeros)
        hid = _gru(agg2, cb, hid, wmats, bvecs)
        h = hid

    return _readout(h, Wr1, br1.reshape(1, RH), Wr2.T.reshape(1, RH),
                    br2.reshape(1, 1))


# speedup vs baseline: 1.1687x; 1.1687x over previous
import functools

import jax
import jax.numpy as jnp
from jax import lax
from jax.experimental import pallas as pl
from jax.experimental.pallas import tpu as pltpu
from jax.experimental.pallas import tpu_sc as plsc

N = 10000
E = 160000
DN = 128
DE = 16
H = 64
EH = 128
RH = 1000
STEPS = 6

NC = 2
NS = 16
NW = NC * NS
CHUNK = 128
EPW = E // NW
CPW = 40
TAIL = EPW - (CPW - 1) * CHUNK
NPAD = N + 8
OUT_ROWS = 640

@functools.cache
def _sc_mesh():
    return plsc.VectorSubcoreMesh(core_axis_name="c", subcore_axis_name="s",
                                  num_cores=NC, num_subcores=NS)


@functools.cache
def _sc_gather_call():
    @functools.partial(
        pl.kernel,
        out_type=jax.ShapeDtypeStruct((E, H), jnp.float32),
        mesh=_sc_mesh(),
        scratch_types=[
            pltpu.VMEM((CPW, CHUNK), jnp.int32),
            pltpu.VMEM((CHUNK, H), jnp.float32),
            pltpu.SemaphoreType.DMA,
        ],
        compiler_params=pltpu.CompilerParams(use_tc_tiling_on_sc=False),
    )
    def _sc_gather(h_hbm, idx_hbm, out_hbm, idx_v, rows_v, sem):
        c = lax.axis_index("c")
        s = lax.axis_index("s")
        wid = s * NC + c
        pltpu.sync_copy(idx_hbm.at[pl.ds(wid * CPW, CPW)], idx_v)
        for j in range(CPW - 1):
            pltpu.async_copy(h_hbm.at[idx_v.at[j]], rows_v, sem).wait()
            pltpu.sync_copy(rows_v,
                            out_hbm.at[pl.ds(wid * EPW + j * CHUNK, CHUNK)])
        pltpu.async_copy(h_hbm.at[idx_v.at[CPW - 1]], rows_v, sem).wait()
        pltpu.sync_copy(
            rows_v.at[pl.ds(0, TAIL)],
            out_hbm.at[pl.ds(wid * EPW + (CPW - 1) * CHUNK, TAIL)])

    return _sc_gather


@functools.cache
def _sc_scatter_call():
    @functools.partial(
        pl.kernel,
        out_type=jax.ShapeDtypeStruct((NC, N, H), jnp.float32),
        mesh=_sc_mesh(),
        scratch_types=[
            pltpu.VMEM((CPW, CHUNK), jnp.int32),
            pltpu.VMEM((CHUNK, H), jnp.float32),
            pltpu.VMEM_SHARED((NPAD, H), jnp.float32),
        ],
        compiler_params=pltpu.CompilerParams(use_tc_tiling_on_sc=False),
    )
    def _sc_scatter(m_hbm, idx_hbm, zeros_hbm, out_hbm, idx_v, m_v, agg_sh):
        c = lax.axis_index("c")
        s = lax.axis_index("s")
        wid = s * NC + c

        @pl.when(s == 0)
        def _():
            pltpu.sync_copy(zeros_hbm, agg_sh)

        plsc.subcore_barrier()
        pltpu.sync_copy(idx_hbm.at[pl.ds(wid * CPW, CPW)], idx_v)
        for j in range(CPW - 1):
            pltpu.sync_copy(m_hbm.at[pl.ds(wid * EPW + j * CHUNK, CHUNK)], m_v)
            pltpu.sync_copy(m_v, agg_sh.at[idx_v.at[j]], add=True)
        pltpu.sync_copy(
            m_hbm.at[pl.ds(wid * EPW + (CPW - 1) * CHUNK, TAIL)],
            m_v.at[pl.ds(0, TAIL)])
        pltpu.sync_copy(m_v, agg_sh.at[idx_v.at[CPW - 1]], add=True)
        plsc.subcore_barrier()
        off = jnp.minimum(s * OUT_ROWS, N - OUT_ROWS)
        pltpu.sync_copy(agg_sh.at[pl.ds(off, OUT_ROWS)],
                        out_hbm.at[c].at[pl.ds(off, OUT_ROWS)])

    return _sc_scatter


def _mm_kernel(x_ref, w_ref, b_ref, o_ref, *, act, out_dtype):
    y = jnp.dot(x_ref[...], w_ref[...], preferred_element_type=jnp.float32, precision=jax.lax.Precision.HIGHEST)
    y = y + b_ref[...]
    if act == "relu":
        y = jnp.maximum(y, 0.0)
    o_ref[...] = y.astype(out_dtype)


def _dense(x, w, b, act, gridn, out_dtype=jnp.float32):
    tm = x.shape[0] // gridn
    return pl.pallas_call(
        functools.partial(_mm_kernel, act=act, out_dtype=out_dtype),
        grid=(gridn,),
        in_specs=[
            pl.BlockSpec((tm, x.shape[1]), lambda i: (i, 0)),
            pl.BlockSpec(w.shape, lambda i: (0, 0)),
            pl.BlockSpec(b.shape, lambda i: (0, 0)),
        ],
        out_specs=pl.BlockSpec((tm, w.shape[1]), lambda i: (i, 0)),
        out_shape=jax.ShapeDtypeStruct((x.shape[0], w.shape[1]), out_dtype),
    )(x, w, b)


TE_MSG = 640


def _msg_kernel(hs_ref, w_ref, o_ref):
    hs = hs_ref[...]
    w = w_ref[...].astype(jnp.float32)
    fa = jnp.broadcast_to(hs[:, :32][:, :, None], (TE_MSG, 32, 64))
    fb = jnp.broadcast_to(hs[:, 32:][:, :, None], (TE_MSG, 32, 64))
    f = jnp.concatenate([fa, fb], axis=2)
    a = jnp.sum(w * f, axis=1)
    o_ref[...] = a[:, :64] + a[:, 64:]


def _messages(hs, wedge3):
    return pl.pallas_call(
        _msg_kernel,
        grid=(E // TE_MSG,),
        in_specs=[
            pl.BlockSpec((TE_MSG, H), lambda i: (i, 0)),
            pl.BlockSpec((TE_MSG, 32, 128), lambda i: (i, 0, 0)),
        ],
        out_specs=pl.BlockSpec((TE_MSG, H), lambda i: (i, 0)),
        out_shape=jax.ShapeDtypeStruct((E, H), jnp.float32),
    )(hs, wedge3)


TM_WEDGE = 1000


def _wedge_kernel(x_ref, w1_ref, b1_ref, w2_ref, b2_ref, o_ref):
    eh = jnp.dot(x_ref[...], w1_ref[...], preferred_element_type=jnp.float32,
                 precision=jax.lax.Precision.HIGHEST)
    eh = jnp.maximum(eh + b1_ref[...], 0.0)
    y = jnp.dot(eh, w2_ref[...], preferred_element_type=jnp.float32,
                precision=jax.lax.Precision.DEFAULT)
    o_ref[...] = (y + b2_ref[...]).astype(jnp.bfloat16)


def _wedge_build(edges, w1, b1, w2, b2):
    return pl.pallas_call(
        _wedge_kernel,
        grid=(E // TM_WEDGE,),
        in_specs=[
            pl.BlockSpec((TM_WEDGE, DE), lambda i: (i, 0)),
            pl.BlockSpec((DE, EH), lambda i: (0, 0)),
            pl.BlockSpec((1, EH), lambda i: (0, 0)),
            pl.BlockSpec((EH, H * H), lambda i: (0, 0)),
            pl.BlockSpec((1, H * H), lambda i: (0, 0)),
        ],
        out_specs=pl.BlockSpec((TM_WEDGE, H * H), lambda i: (i, 0)),
        out_shape=jax.ShapeDtypeStruct((E, H * H), jnp.bfloat16),
    )(edges, w1, b1, w2, b2)


def _gru_kernel(agg_ref, cb_ref, hid_ref, wir_ref, wiz_ref, win_ref,
                whr_ref, whz_ref, whn_ref, bir_ref, biz_ref, bin_ref,
                bhr_ref, bhz_ref, bhn_ref, o_ref):
    a = jnp.maximum(agg_ref[0] + agg_ref[1] + cb_ref[...], 0.0)
    hid = hid_ref[...]

    def mm(x, wref):
        return jnp.dot(x, wref[...], preferred_element_type=jnp.float32, precision=jax.lax.Precision.HIGHEST)

    r = jax.nn.sigmoid(mm(a, wir_ref) + bir_ref[...] + mm(hid, whr_ref)
                       + bhr_ref[...])
    z = jax.nn.sigmoid(mm(a, wiz_ref) + biz_ref[...] + mm(hid, whz_ref)
                       + bhz_ref[...])
    n = jnp.tanh(mm(a, win_ref) + bin_ref[...]
                 + r * (mm(hid, whn_ref) + bhn_ref[...]))
    o_ref[...] = (1.0 - z) * n + z * hid


TN_GRU = 1000


def _gru(agg2, cb, hid, wmats, bvecs):
    full = lambda shape: pl.BlockSpec(shape, lambda i: (0,) * len(shape))
    return pl.pallas_call(
        _gru_kernel,
        grid=(N // TN_GRU,),
        in_specs=[
            pl.BlockSpec((NC, TN_GRU, H), lambda i: (0, i, 0)),
            full((1, H)),
            pl.BlockSpec((TN_GRU, H), lambda i: (i, 0)),
        ] + [full((H, H))] * 6 + [full((1, H))] * 6,
        out_specs=pl.BlockSpec((TN_GRU, H), lambda i: (i, 0)),
        out_shape=jax.ShapeDtypeStruct((N, H), jnp.float32),
    )(agg2, cb, hid, *wmats, *bvecs)


def _readout_kernel(h_ref, w1_ref, b1_ref, w2r_ref, b2_ref, o_ref):
    i = pl.program_id(0)
    x = jnp.dot(h_ref[...], w1_ref[...], preferred_element_type=jnp.float32, precision=jax.lax.Precision.HIGHEST)
    x = x + b1_ref[...]
    x = jnp.where(x > 0, x, jnp.exp(jnp.minimum(x, 0.0)) - 1.0)
    part = jnp.sum(x * w2r_ref[...]) + TN_GRU * jnp.sum(b2_ref[...])

    @pl.when(i == 0)
    def _():
        o_ref[...] = jnp.zeros((1, 1), jnp.float32)

    o_ref[...] += jnp.full((1, 1), part)


def _readout(h, w1, b1, w2r, b2):
    full = lambda shape: pl.BlockSpec(shape, lambda i: (0,) * len(shape))
    return pl.pallas_call(
        _readout_kernel,
        grid=(N // TN_GRU,),
        in_specs=[
            pl.BlockSpec((TN_GRU, H), lambda i: (i, 0)),
            full((H, RH)),
            full((1, RH)),
            full((1, RH)),
            full((1, 1)),
        ],
        out_specs=pl.BlockSpec((1, 1), lambda i: (0, 0)),
        out_shape=jax.ShapeDtypeStruct((1, 1), jnp.float32),
    )(h, w1, b1, w2r, b2)


def kernel(edge_index, nodes, edges, W0, b0, We1, be1, We2, be2, conv_bias,
           W_ih, W_hh, b_ih, b_hh, Wr1, br1, Wr2, br2):
    f32 = jnp.float32
    padc = CPW * CHUNK - EPW
    src = jnp.pad(edge_index[0].reshape(NW, EPW), ((0, 0), (0, padc)),
                  constant_values=0).reshape(NW * CPW, CHUNK)
    dst = jnp.pad(edge_index[1].reshape(NW, EPW), ((0, 0), (0, padc)),
                  constant_values=N).reshape(NW * CPW, CHUNK)

    h0 = _dense(nodes, W0, b0.reshape(1, H), "relu", 10)
    j = jnp.arange(H * H)
    p_idx = j // 128
    q_idx = j % 128
    perm = (p_idx + 32 * (q_idx >= 64)) * H + (q_idx % 64)
    We2p = We2[:, perm]
    be2p = be2[perm]
    wedge2d = _wedge_build(edges, We1, be1.reshape(1, EH), We2p,
                           be2p.reshape(1, H * H))
    wedge3 = wedge2d.reshape(E, 32, 128)

    zeros = jnp.zeros((NPAD, H), f32)
    cb = conv_bias.reshape(1, H)
    wmats = [W_ih[:H].T, W_ih[H:2 * H].T, W_ih[2 * H:].T,
             W_hh[:H].T, W_hh[H:2 * H].T, W_hh[2 * H:].T]
    bvecs = [b_ih[:H].reshape(1, H), b_ih[H:2 * H].reshape(1, H),
             b_ih[2 * H:].reshape(1, H), b_hh[:H].reshape(1, H),
             b_hh[H:2 * H].reshape(1, H), b_hh[2 * H:].reshape(1, H)]

    h = h0
    hid = h0
    for _ in range(STEPS):
        hs = _sc_gather_call()(h, src)
        m = _messages(hs, wedge3)
        agg2 = _sc_scatter_call()(m, dst, zeros)
        hid = _gru(agg2, cb, hid, wmats, bvecs)
        h = hid

    return _readout(h, Wr1, br1.reshape(1, RH), Wr2.T.reshape(1, RH),
                    br2.reshape(1, 1))
